# Initial kernel scaffold; baseline (speedup 1.0000x reference)
import jax
import jax.numpy as jnp
from jax.experimental import pallas as pl


def kernel(x, edge_index, A_wave, W_gat, att_src, att_dst, bias_gat, W_ih, W_hh, b_ih, b_hh, W1, b1, W2, b2):
    raise NotImplementedError("write your pallas kernel here")


# speedup vs baseline: 24.7323x; 24.7323x over previous
import jax
import jax.numpy as jnp
from jax import lax
from jax.experimental import pallas as pl
from jax.experimental.pallas import tpu as pltpu
from jax.experimental.pallas import tpu_sc as plsc

N = 10000
E = 160000
EA = E + N
HIS = 12
HID = 16
HEADS = 3
OUT_F = 192
F = 576
HF = 640
PRED = 6

NC = 2
NS = 16
NW = NC * NS
CPT = 334
EPT = CPT * 16
EPAD = NW * EPT
ACCF = 30720
RPT = ACCF // NS

_f32 = jnp.float32
_bf16 = jnp.bfloat16
_i32 = jnp.int32

_SC_PARAMS = pltpu.CompilerParams(
    needs_layout_passes=False, use_tc_tiling_on_sc=False)


def _sc_mesh():
    return plsc.VectorSubcoreMesh(core_axis_name="c", subcore_axis_name="s")


def _k1_body(x_ref, w_ref, as_ref, ad_ref, h_ref, logit_ref):
    h = jnp.dot(x_ref[...], w_ref[...], preferred_element_type=_f32)
    h_ref[...] = jnp.concatenate(
        [h, jnp.zeros((h.shape[0], HF - F), _f32)], axis=1)
    h3 = h.reshape(h.shape[0], HEADS, OUT_F)
    a_s = jnp.sum(h3 * as_ref[...][None], axis=-1)
    a_d = jnp.sum(h3 * ad_ref[...][None], axis=-1)
    logit_ref[...] = jnp.concatenate([a_s, a_d], axis=-1)


def _k1(xf, w_gat, att_s, att_d):
    blk = 1000
    return pl.pallas_call(
        _k1_body,
        grid=(N // blk,),
        in_specs=[
            pl.BlockSpec((blk, 96), lambda i: (i, 0)),
            pl.BlockSpec((96, F), lambda i: (0, 0)),
            pl.BlockSpec((HEADS, OUT_F), lambda i: (0, 0)),
            pl.BlockSpec((HEADS, OUT_F), lambda i: (0, 0)),
        ],
        out_specs=[
            pl.BlockSpec((blk, HF), lambda i: (i, 0)),
            pl.BlockSpec((blk, 6), lambda i: (i, 0)),
        ],
        out_shape=[
            jax.ShapeDtypeStruct((N, HF), _f32),
            jax.ShapeDtypeStruct((N, 6), _f32),
        ],
    )(xf, w_gat, att_s, att_d)


def _k2_body(src_hbm, dst_hbm, logit_hbm, wt_hbm, parts_hbm,
             srcv, dstv, adt, acc, wbuf):
    cid = lax.axis_index("c")
    sid = lax.axis_index("s")
    wid = sid * NC + cid
    base = wid * EPT

    pltpu.sync_copy(src_hbm.at[pl.ds(base, EPT)], srcv)
    pltpu.sync_copy(dst_hbm.at[pl.ds(base, EPT)], dstv)
    pltpu.sync_copy(logit_hbm, adt)

    def _zero(i, _):
        acc[pl.ds(i * 16, 16)] = jnp.zeros((16,), _f32)
        return _
    lax.fori_loop(0, ACCF // 16, _zero, None)

    iota = lax.iota(_i32, 16)

    def _chunk(c, _):
        off = c * 16
        s16 = srcv[pl.ds(off, 16)]
        d16 = dstv[pl.ds(off, 16)]
        valid = (base + off + iota) < EA
        s6 = s16 * 6
        d6 = d16 * 6
        for hh in range(HEADS):
            a = plsc.load_gather(adt, [s6 + hh]) + \
                plsc.load_gather(adt, [d6 + (hh + 3)])
            a = jnp.where(a > 0, a, a * 0.2)
            w = jnp.where(valid, jnp.exp(a), 0.0)
            wbuf[pl.ds(hh * EPT + off, 16)] = w
            plsc.addupdate_scatter(acc, [d16 * 3 + hh], w)
        return _
    lax.fori_loop(0, CPT, _chunk, None)

    for hh in range(HEADS):
        pltpu.sync_copy(wbuf.at[pl.ds(hh * EPT, EPT)],
                        wt_hbm.at[pl.ds(hh * EPAD + base, EPT)])
    pltpu.sync_copy(acc, parts_hbm.at[pl.ds(wid * ACCF, ACCF)])


def _k2(srcp, dstp, logit):
    fn = pl.kernel(
        _k2_body,
        out_type=[
            jax.ShapeDtypeStruct((HEADS * EPAD,), _f32),
            jax.ShapeDtypeStruct((NW * ACCF,), _f32),
        ],
        mesh=_sc_mesh(),
        scratch_types=[
            pltpu.VMEM((EPT,), _i32),
            pltpu.VMEM((EPT,), _i32),
            pltpu.VMEM((N * 6,), _f32),
            pltpu.VMEM((ACCF,), _f32),
            pltpu.VMEM((HEADS * EPT,), _f32),
        ],
        compiler_params=_SC_PARAMS,
    )
    return fn(srcp, dstp, logit)


def _k2b_body(dst_hbm, wt_hbm, parts_hbm, ct_hbm,
              dstv, wv, rdt, dsum, tmp, rdenS):
    cid = lax.axis_index("c")
    sid = lax.axis_index("s")
    wid = sid * NC + cid
    base = wid * EPT

    pltpu.sync_copy(dst_hbm.at[pl.ds(base, EPT)], dstv)
    for hh in range(HEADS):
        pltpu.sync_copy(wt_hbm.at[pl.ds(hh * EPAD + base, EPT)],
                        wv.at[pl.ds(hh * EPT, EPT)])

    fr = sid * RPT
    pltpu.sync_copy(parts_hbm.at[pl.ds(fr, RPT)], dsum)

    def _accum_part(w, _):
        pltpu.sync_copy(parts_hbm.at[pl.ds(w * ACCF + fr, RPT)], tmp)

        def _add(t, __):
            so = pl.ds(t * 16, 16)
            dsum[so] = dsum[so] + tmp[so]
            return __
        lax.fori_loop(0, RPT // 16, _add, None)
        return _
    lax.fori_loop(1, NW, _accum_part, None)

    def _recip(t, _):
        so = pl.ds(t * 16, 16)
        dsum[so] = 1.0 / (dsum[so] + 1e-16)
        return _
    lax.fori_loop(0, RPT // 16, _recip, None)
    pltpu.sync_copy(dsum, rdenS.at[pl.ds(fr, RPT)])
    plsc.subcore_barrier()
    pltpu.sync_copy(rdenS, rdt)

    third = 1.0 / HEADS

    def _chunk(c, _):
        off = c * 16
        d16 = dstv[pl.ds(off, 16)]
        d3 = d16 * 3
        for hh in range(HEADS):
            sl = pl.ds(hh * EPT + off, 16)
            r16 = plsc.load_gather(rdt, [d3 + hh])
            wv[sl] = wv[sl] * r16 * third
        return _
    lax.fori_loop(0, CPT, _chunk, None)

    for hh in range(HEADS):
        pltpu.sync_copy(wv.at[pl.ds(hh * EPT, EPT)],
                        ct_hbm.at[pl.ds(hh * EPAD + base, EPT)])


def _k2b(dstp, wt, parts):
    fn = pl.kernel(
        _k2b_body,
        out_type=jax.ShapeDtypeStruct((HEADS * EPAD,), _f32),
        mesh=_sc_mesh(),
        scratch_types=[
            pltpu.VMEM((EPT,), _i32),
            pltpu.VMEM((HEADS * EPT,), _f32),
            pltpu.VMEM((ACCF,), _f32),
            pltpu.VMEM((RPT,), _f32),
            pltpu.VMEM((RPT,), _f32),
            pltpu.MemorySpace.VMEM_SHARED((ACCF,), _f32),
        ],
        compiler_params=_SC_PARAMS,
    )
    return fn(dstp, wt, parts)


def _k3_body(src_hbm, dst_hbm, ct_hbm, h_hbm, outp_hbm,
             srcv, dstv, ctv, rows, msg, accum, gsem0, gsem1):
    cid = lax.axis_index("c")
    sid = lax.axis_index("s")
    wid = sid * NC + cid
    base = wid * EPT

    pltpu.sync_copy(src_hbm.at[pl.ds(base, EPT)], srcv)
    pltpu.sync_copy(dst_hbm.at[pl.ds(base, EPT)], dstv)
    for hh in range(HEADS):
        pltpu.sync_copy(ct_hbm.at[pl.ds(hh * EPAD + base, EPT)],
                        ctv.at[pl.ds(hh * EPT, EPT)])

    for j in range(16):
        for q in range(OUT_F // 32):
            z = jnp.zeros((32,), _bf16)
            msg[j, pl.ds(q * 32, 32)] = z
    nrows = 624
    for r in range(nrows // 16):
        pltpu.sync_copy(msg, accum.at[pl.ds(sid * nrows + r * 16, 16)])

    @pl.when(sid == NS - 1)
    def _zero_tail():
        pltpu.sync_copy(msg, accum.at[pl.ds(NS * nrows, 16)])

    plsc.subcore_barrier()

    def _chunk(c, b):
        gsem = gsem0 if b == 0 else gsem1
        s16 = srcv[pl.ds(c * 16, 16)]
        pltpu.make_async_copy(h_hbm.at[s16], rows.at[b], gsem).wait()
        d16 = dstv[pl.ds(c * 16, 16)]
        cv = [ctv[pl.ds(hh * EPT + c * 16, 16)] for hh in range(HEADS)]
        for j in range(16):
            c0 = cv[0][j]
            c1 = cv[1][j]
            c2 = cv[2][j]
            for q in range(OUT_F // 32):
                m0 = (rows[b, j, pl.ds(q * 32, 16)] * c0
                      + rows[b, j, pl.ds(OUT_F + q * 32, 16)] * c1
                      + rows[b, j, pl.ds(2 * OUT_F + q * 32, 16)] * c2)
                m1 = (rows[b, j, pl.ds(q * 32 + 16, 16)] * c0
                      + rows[b, j, pl.ds(OUT_F + q * 32 + 16, 16)] * c1
                      + rows[b, j, pl.ds(2 * OUT_F + q * 32 + 16, 16)] * c2)
                msg[j, pl.ds(q * 32, 32)] = plsc.pack(
                    m0, m1, format=plsc.PackFormat.INTERLEAVED)

        @pl.when(c + 2 < CPT)
        def _issue_next():
            s16n = srcv[pl.ds((c + 2) * 16, 16)]
            pltpu.async_copy(h_hbm.at[s16n], rows.at[b], gsem)

        pltpu.sync_copy(msg, accum.at[d16], add=True)

    s0 = srcv[pl.ds(0, 16)]
    pltpu.async_copy(h_hbm.at[s0], rows.at[0], gsem0)
    s1 = srcv[pl.ds(16, 16)]
    pltpu.async_copy(h_hbm.at[s1], rows.at[1], gsem1)

    def _pair(i, _):
        _chunk(2 * i, 0)
        _chunk(2 * i + 1, 1)
        return _
    lax.fori_loop(0, CPT // 2, _pair, None)

    plsc.subcore_barrier()
    pltpu.sync_copy(accum.at[pl.ds(sid * nrows, nrows)],
                    outp_hbm.at[pl.ds(cid * N + sid * nrows, nrows)])

    @pl.when(sid == NS - 1)
    def _out_tail():
        pltpu.sync_copy(accum.at[pl.ds(NS * nrows, 16)],
                        outp_hbm.at[pl.ds(cid * N + NS * nrows, 16)])


def _k3(srcp, dstp, ct, h):
    fn = pl.kernel(
        _k3_body,
        out_type=jax.ShapeDtypeStruct((NC * N, OUT_F), _bf16),
        mesh=_sc_mesh(),
        scratch_types=[
            pltpu.VMEM((EPT,), _i32),
            pltpu.VMEM((EPT,), _i32),
            pltpu.VMEM((HEADS * EPT,), _f32),
            pltpu.VMEM((2, 16, HF), _f32),
            pltpu.VMEM((16, OUT_F), _bf16),
            pltpu.MemorySpace.VMEM_SHARED((N, OUT_F), _bf16),
            pltpu.SemaphoreType.DMA,
            pltpu.SemaphoreType.DMA,
        ],
        compiler_params=_SC_PARAMS,
    )
    return fn(srcp, dstp, ct, h)


def _k4_body(outp_ref, bias_ref, wih_ref, whh_ref, bih_ref, bhh_ref,
             w1_ref, b1_ref, w2t_ref, b2_ref, out_ref):
    g = outp_ref[0] + outp_ref[1] + bias_ref[...]
    nb = g.shape[0]
    hprev = jnp.zeros((nb, HID), _f32)
    acc = jnp.zeros((nb, PRED), _f32)
    for t in range(HIS):
        xt = g[:, t * HID:(t + 1) * HID]
        gi = jnp.dot(xt, wih_ref[...], preferred_element_type=_f32) + bih_ref[...]
        gh = jnp.dot(hprev, whh_ref[...], preferred_element_type=_f32) + bhh_ref[...]
        r = jax.nn.sigmoid(gi[:, :HID] + gh[:, :HID])
        z = jax.nn.sigmoid(gi[:, HID:2 * HID] + gh[:, HID:2 * HID])
        nn_ = jnp.tanh(gi[:, 2 * HID:] + r * gh[:, 2 * HID:])
        hprev = (1.0 - z) * nn_ + z * hprev
        o1 = jnp.sum(hprev * w1_ref[...], axis=1, keepdims=True) + b1_ref[...]
        acc = acc + o1 * w2t_ref[t:t + 1, :]
    out_ref[...] = acc + b2_ref[...]


def _k4(outp, bias, wih_t, whh_t, bih, bhh, w1, b1, w2t, b2):
    blk = 2000
    return pl.pallas_call(
        _k4_body,
        grid=(N // blk,),
        in_specs=[
            pl.BlockSpec((NC, blk, OUT_F), lambda i: (0, i, 0)),
            pl.BlockSpec((1, OUT_F), lambda i: (0, 0)),
            pl.BlockSpec((HID, 3 * HID), lambda i: (0, 0)),
            pl.BlockSpec((HID, 3 * HID), lambda i: (0, 0)),
            pl.BlockSpec((1, 3 * HID), lambda i: (0, 0)),
            pl.BlockSpec((1, 3 * HID), lambda i: (0, 0)),
            pl.BlockSpec((1, HID), lambda i: (0, 0)),
            pl.BlockSpec((1, 1), lambda i: (0, 0)),
            pl.BlockSpec((HIS, PRED), lambda i: (0, 0)),
            pl.BlockSpec((1, PRED), lambda i: (0, 0)),
        ],
        out_specs=pl.BlockSpec((blk, PRED), lambda i: (i, 0)),
        out_shape=jax.ShapeDtypeStruct((N, PRED), _f32),
    )(outp, bias, wih_t, whh_t, bih, bhh, w1, b1, w2t, b2)


def kernel(x, edge_index, A_wave, W_gat, att_src, att_dst, bias_gat,
           W_ih, W_hh, b_ih, b_hh, W1, b1, W2, b2):
    xf = x.reshape(N, 96)
    loop = jnp.arange(N, dtype=edge_index.dtype)
    padz = jnp.zeros((EPAD - EA,), edge_index.dtype)
    srcp = jnp.concatenate([edge_index[0], loop, padz])
    dstp = jnp.concatenate([edge_index[1], loop, padz])

    h, logit = _k1(xf, W_gat,
                   att_src.reshape(HEADS, OUT_F),
                   att_dst.reshape(HEADS, OUT_F))
    wt, parts = _k2(srcp, dstp, logit.reshape(N * 6))
    ct = _k2b(dstp, wt, parts)
    outp = _k3(srcp, dstp, ct, h)

    outp = outp.reshape(NC, N, OUT_F // 32, 16, 2)
    outp = jnp.swapaxes(outp, 3, 4).reshape(NC, N, OUT_F).astype(_f32)

    out = _k4(outp, bias_gat.reshape(1, OUT_F),
              W_ih.T, W_hh.T, b_ih.reshape(1, 3 * HID), b_hh.reshape(1, 3 * HID),
              W1, b1.reshape(1, 1), W2.T, b2.reshape(1, PRED))
    return (out.reshape(1, N, PRED), A_wave)


# speedup vs baseline: 37.7610x; 1.5268x over previous
import jax
import jax.numpy as jnp
from jax import lax
from jax.experimental import pallas as pl
from jax.experimental.pallas import tpu as pltpu
from jax.experimental.pallas import tpu_sc as plsc

N = 10000
E = 160000
EA = E + N
HIS = 12
HID = 16
HEADS = 3
OUT_F = 192
F = 576
HF = 640
PRED = 6

NC = 2
NS = 16
NW = NC * NS
CPT = 334
EPT = CPT * 16
EPAD = NW * EPT
ACCF = 30720
RPT = ACCF // NS

_f32 = jnp.float32
_bf16 = jnp.bfloat16
_i32 = jnp.int32

_SC_PARAMS = pltpu.CompilerParams(
    needs_layout_passes=False, use_tc_tiling_on_sc=False)


def _sc_mesh():
    return plsc.VectorSubcoreMesh(core_axis_name="c", subcore_axis_name="s")


def _k1_body(x_ref, w_ref, as_ref, ad_ref, h_ref, logit_ref):
    h = jnp.dot(x_ref[...], w_ref[...], preferred_element_type=_f32)
    h_ref[...] = h.astype(_bf16)
    h3 = h.reshape(h.shape[0], HEADS, OUT_F)
    a_s = jnp.sum(h3 * as_ref[...][None], axis=-1)
    a_d = jnp.sum(h3 * ad_ref[...][None], axis=-1)
    logit_ref[...] = jnp.concatenate([a_s, a_d], axis=-1)


def _k1(xf, w_gat, att_s, att_d):
    blk = 1000
    return pl.pallas_call(
        _k1_body,
        grid=(N // blk,),
        in_specs=[
            pl.BlockSpec((blk, 96), lambda i: (i, 0)),
            pl.BlockSpec((96, F), lambda i: (0, 0)),
            pl.BlockSpec((HEADS, OUT_F), lambda i: (0, 0)),
            pl.BlockSpec((HEADS, OUT_F), lambda i: (0, 0)),
        ],
        out_specs=[
            pl.BlockSpec((blk, F), lambda i: (i, 0)),
            pl.BlockSpec((blk, 6), lambda i: (i, 0)),
        ],
        out_shape=[
            jax.ShapeDtypeStruct((N, F), _bf16),
            jax.ShapeDtypeStruct((N, 6), _f32),
        ],
    )(xf, w_gat, att_s, att_d)


def _k2_body(src_hbm, dst_hbm, logit_hbm, wt_hbm, parts_hbm,
             srcv, dstv, adt, acc, wbuf):
    cid = lax.axis_index("c")
    sid = lax.axis_index("s")
    wid = sid * NC + cid
    base = wid * EPT

    pltpu.sync_copy(src_hbm.at[pl.ds(base, EPT)], srcv)
    pltpu.sync_copy(dst_hbm.at[pl.ds(base, EPT)], dstv)
    pltpu.sync_copy(logit_hbm, adt)

    def _zero(i, _):
        acc[pl.ds(i * 16, 16)] = jnp.zeros((16,), _f32)
        return _
    lax.fori_loop(0, ACCF // 16, _zero, None)

    iota = lax.iota(_i32, 16)

    def _chunk(c, _):
        off = c * 16
        s16 = srcv[pl.ds(off, 16)]
        d16 = dstv[pl.ds(off, 16)]
        valid = (base + off + iota) < EA
        s6 = s16 * 6
        d6 = d16 * 6
        for hh in range(HEADS):
            a = plsc.load_gather(adt, [s6 + hh]) + \
                plsc.load_gather(adt, [d6 + (hh + 3)])
            a = jnp.where(a > 0, a, a * 0.2)
            w = jnp.where(valid, jnp.exp(a), 0.0)
            wbuf[pl.ds(hh * EPT + off, 16)] = w
            plsc.addupdate_scatter(acc, [d16 * 3 + hh], w)
        return _
    lax.fori_loop(0, CPT, _chunk, None)

    for hh in range(HEADS):
        pltpu.sync_copy(wbuf.at[pl.ds(hh * EPT, EPT)],
                        wt_hbm.at[pl.ds(hh * EPAD + base, EPT)])
    pltpu.sync_copy(acc, parts_hbm.at[pl.ds(wid * ACCF, ACCF)])


def _k2(srcp, dstp, logit):
    fn = pl.kernel(
        _k2_body,
        out_type=[
            jax.ShapeDtypeStruct((HEADS * EPAD,), _f32),
            jax.ShapeDtypeStruct((NW * ACCF,), _f32),
        ],
        mesh=_sc_mesh(),
        scratch_types=[
            pltpu.VMEM((EPT,), _i32),
            pltpu.VMEM((EPT,), _i32),
            pltpu.VMEM((N * 6,), _f32),
            pltpu.VMEM((ACCF,), _f32),
            pltpu.VMEM((HEADS * EPT,), _f32),
        ],
        compiler_params=_SC_PARAMS,
    )
    return fn(srcp, dstp, logit)


def _k2b_body(dst_hbm, wt_hbm, parts_hbm, ct_hbm,
              dstv, wv, rdt, dsum, tmp, rdenS):
    cid = lax.axis_index("c")
    sid = lax.axis_index("s")
    wid = sid * NC + cid
    base = wid * EPT

    pltpu.sync_copy(dst_hbm.at[pl.ds(base, EPT)], dstv)
    for hh in range(HEADS):
        pltpu.sync_copy(wt_hbm.at[pl.ds(hh * EPAD + base, EPT)],
                        wv.at[pl.ds(hh * EPT, EPT)])

    fr = sid * RPT
    pltpu.sync_copy(parts_hbm.at[pl.ds(fr, RPT)], dsum)

    def _accum_part(w, _):
        pltpu.sync_copy(parts_hbm.at[pl.ds(w * ACCF + fr, RPT)], tmp)

        def _add(t, __):
            so = pl.ds(t * 16, 16)
            dsum[so] = dsum[so] + tmp[so]
            return __
        lax.fori_loop(0, RPT // 16, _add, None)
        return _
    lax.fori_loop(1, NW, _accum_part, None)

    def _recip(t, _):
        so = pl.ds(t * 16, 16)
        dsum[so] = 1.0 / (dsum[so] + 1e-16)
        return _
    lax.fori_loop(0, RPT // 16, _recip, None)
    pltpu.sync_copy(dsum, rdenS.at[pl.ds(fr, RPT)])
    plsc.subcore_barrier()
    pltpu.sync_copy(rdenS, rdt)

    third = 1.0 / HEADS

    def _chunk(c, _):
        off = c * 16
        d16 = dstv[pl.ds(off, 16)]
        d3 = d16 * 3
        for hh in range(HEADS):
            sl = pl.ds(hh * EPT + off, 16)
            r16 = plsc.load_gather(rdt, [d3 + hh])
            wv[sl] = wv[sl] * r16 * third
        return _
    lax.fori_loop(0, CPT, _chunk, None)

    for hh in range(HEADS):
        pltpu.sync_copy(wv.at[pl.ds(hh * EPT, EPT)],
                        ct_hbm.at[pl.ds(hh * EPAD + base, EPT)])


def _k2b(dstp, wt, parts):
    fn = pl.kernel(
        _k2b_body,
        out_type=jax.ShapeDtypeStruct((HEADS * EPAD,), _f32),
        mesh=_sc_mesh(),
        scratch_types=[
            pltpu.VMEM((EPT,), _i32),
            pltpu.VMEM((HEADS * EPT,), _f32),
            pltpu.VMEM((ACCF,), _f32),
            pltpu.VMEM((RPT,), _f32),
            pltpu.VMEM((RPT,), _f32),
            pltpu.MemorySpace.VMEM_SHARED((ACCF,), _f32),
        ],
        compiler_params=_SC_PARAMS,
    )
    return fn(dstp, wt, parts)


def _k3_body(src_hbm, dst_hbm, ct_hbm, h_hbm, outp_hbm,
             srcv, dstv, ctv, rows, msg, accum,
             gsem0, gsem1, ssem0, ssem1):
    cid = lax.axis_index("c")
    sid = lax.axis_index("s")
    wid = sid * NC + cid
    base = wid * EPT

    pltpu.sync_copy(src_hbm.at[pl.ds(base, EPT)], srcv)
    pltpu.sync_copy(dst_hbm.at[pl.ds(base, EPT)], dstv)
    for hh in range(HEADS):
        pltpu.sync_copy(ct_hbm.at[pl.ds(hh * EPAD + base, EPT)],
                        ctv.at[pl.ds(hh * EPT, EPT)])

    for j in range(16):
        for q in range(OUT_F // 32):
            z = jnp.zeros((32,), _bf16)
            msg[0, j, pl.ds(q * 32, 32)] = z
    nrows = 624
    for r in range(nrows // 16):
        pltpu.sync_copy(msg.at[0], accum.at[pl.ds(sid * nrows + r * 16, 16)])

    @pl.when(sid == NS - 1)
    def _zero_tail():
        pltpu.sync_copy(msg.at[0], accum.at[pl.ds(NS * nrows, 16)])

    plsc.subcore_barrier()

    def _chunk(c, b):
        gsem = gsem0 if b == 0 else gsem1
        ssem = ssem0 if b == 0 else ssem1
        s16 = srcv[pl.ds(c * 16, 16)]
        pltpu.make_async_copy(h_hbm.at[s16], rows.at[b], gsem).wait()
        d16 = dstv[pl.ds(c * 16, 16)]

        @pl.when(c >= 2)
        def _drain_prev_scatter():
            pltpu.make_async_copy(
                msg.at[b], accum.at[pl.ds(0, 16)], ssem).wait()

        cv = [ctv[pl.ds(hh * EPT + c * 16, 16)] for hh in range(HEADS)]
        for j in range(16):
            c0 = cv[0][j]
            c1 = cv[1][j]
            c2 = cv[2][j]
            for q in range(OUT_F // 32):
                r0e, r0o = plsc.unpack(
                    rows[b, j, pl.ds(q * 32, 32)],
                    format=plsc.PackFormat.INTERLEAVED)
                r1e, r1o = plsc.unpack(
                    rows[b, j, pl.ds(OUT_F + q * 32, 32)],
                    format=plsc.PackFormat.INTERLEAVED)
                r2e, r2o = plsc.unpack(
                    rows[b, j, pl.ds(2 * OUT_F + q * 32, 32)],
                    format=plsc.PackFormat.INTERLEAVED)
                me = r0e * c0 + r1e * c1 + r2e * c2
                mo = r0o * c0 + r1o * c1 + r2o * c2
                msg[b, j, pl.ds(q * 32, 32)] = plsc.pack(
                    me, mo, format=plsc.PackFormat.INTERLEAVED)

        @pl.when(c + 2 < CPT)
        def _issue_next():
            s16n = srcv[pl.ds((c + 2) * 16, 16)]
            pltpu.async_copy(h_hbm.at[s16n], rows.at[b], gsem)

        pltpu.async_copy(msg.at[b], accum.at[d16], ssem, add=True)

    s0 = srcv[pl.ds(0, 16)]
    pltpu.async_copy(h_hbm.at[s0], rows.at[0], gsem0)
    s1 = srcv[pl.ds(16, 16)]
    pltpu.async_copy(h_hbm.at[s1], rows.at[1], gsem1)

    def _pair(i, _):
        _chunk(2 * i, 0)
        _chunk(2 * i + 1, 1)
        return _
    lax.fori_loop(0, CPT // 2, _pair, None)

    pltpu.make_async_copy(msg.at[0], accum.at[pl.ds(0, 16)], ssem0).wait()
    pltpu.make_async_copy(msg.at[1], accum.at[pl.ds(0, 16)], ssem1).wait()

    plsc.subcore_barrier()
    pltpu.sync_copy(accum.at[pl.ds(sid * nrows, nrows)],
                    outp_hbm.at[pl.ds(cid * N + sid * nrows, nrows)])

    @pl.when(sid == NS - 1)
    def _out_tail():
        pltpu.sync_copy(accum.at[pl.ds(NS * nrows, 16)],
                        outp_hbm.at[pl.ds(cid * N + NS * nrows, 16)])


def _k3(srcp, dstp, ct, h):
    fn = pl.kernel(
        _k3_body,
        out_type=jax.ShapeDtypeStruct((NC * N, OUT_F), _bf16),
        mesh=_sc_mesh(),
        scratch_types=[
            pltpu.VMEM((EPT,), _i32),
            pltpu.VMEM((EPT,), _i32),
            pltpu.VMEM((HEADS * EPT,), _f32),
            pltpu.VMEM((2, 16, F), _bf16),
            pltpu.VMEM((2, 16, OUT_F), _bf16),
            pltpu.MemorySpace.VMEM_SHARED((N, OUT_F), _bf16),
            pltpu.SemaphoreType.DMA,
            pltpu.SemaphoreType.DMA,
            pltpu.SemaphoreType.DMA,
            pltpu.SemaphoreType.DMA,
        ],
        compiler_params=_SC_PARAMS,
    )
    return fn(srcp, dstp, ct, h)


def _k4_body(outp_ref, bias_ref, wih_ref, whh_ref, bih_ref, bhh_ref,
             w1_ref, b1_ref, w2t_ref, b2_ref, out_ref):
    g = outp_ref[0] + outp_ref[1] + bias_ref[...]
    nb = g.shape[0]
    hprev = jnp.zeros((nb, HID), _f32)
    acc = jnp.zeros((nb, PRED), _f32)
    for t in range(HIS):
        xt = g[:, t * HID:(t + 1) * HID]
        gi = jnp.dot(xt, wih_ref[...], preferred_element_type=_f32) + bih_ref[...]
        gh = jnp.dot(hprev, whh_ref[...], preferred_element_type=_f32) + bhh_ref[...]
        r = jax.nn.sigmoid(gi[:, :HID] + gh[:, :HID])
        z = jax.nn.sigmoid(gi[:, HID:2 * HID] + gh[:, HID:2 * HID])
        nn_ = jnp.tanh(gi[:, 2 * HID:] + r * gh[:, 2 * HID:])
        hprev = (1.0 - z) * nn_ + z * hprev
        o1 = jnp.sum(hprev * w1_ref[...], axis=1, keepdims=True) + b1_ref[...]
        acc = acc + o1 * w2t_ref[t:t + 1, :]
    out_ref[...] = acc + b2_ref[...]


def _k4(outp, bias, wih_t, whh_t, bih, bhh, w1, b1, w2t, b2):
    blk = 2000
    return pl.pallas_call(
        _k4_body,
        grid=(N // blk,),
        in_specs=[
            pl.BlockSpec((NC, blk, OUT_F), lambda i: (0, i, 0)),
            pl.BlockSpec((1, OUT_F), lambda i: (0, 0)),
            pl.BlockSpec((HID, 3 * HID), lambda i: (0, 0)),
            pl.BlockSpec((HID, 3 * HID), lambda i: (0, 0)),
            pl.BlockSpec((1, 3 * HID), lambda i: (0, 0)),
            pl.BlockSpec((1, 3 * HID), lambda i: (0, 0)),
            pl.BlockSpec((1, HID), lambda i: (0, 0)),
            pl.BlockSpec((1, 1), lambda i: (0, 0)),
            pl.BlockSpec((HIS, PRED), lambda i: (0, 0)),
            pl.BlockSpec((1, PRED), lambda i: (0, 0)),
        ],
        out_specs=pl.BlockSpec((blk, PRED), lambda i: (i, 0)),
        out_shape=jax.ShapeDtypeStruct((N, PRED), _f32),
    )(outp, bias, wih_t, whh_t, bih, bhh, w1, b1, w2t, b2)


def kernel(x, edge_index, A_wave, W_gat, att_src, att_dst, bias_gat,
           W_ih, W_hh, b_ih, b_hh, W1, b1, W2, b2):
    xf = x.reshape(N, 96)
    loop = jnp.arange(N, dtype=edge_index.dtype)
    padz = jnp.zeros((EPAD - EA,), edge_index.dtype)
    srcp = jnp.concatenate([edge_index[0], loop, padz])
    dstp = jnp.concatenate([edge_index[1], loop, padz])

    h, logit = _k1(xf, W_gat,
                   att_src.reshape(HEADS, OUT_F),
                   att_dst.reshape(HEADS, OUT_F))
    wt, parts = _k2(srcp, dstp, logit.reshape(N * 6))
    ct = _k2b(dstp, wt, parts)
    outp = _k3(srcp, dstp, ct, h)
    outp = outp.reshape(NC, N, OUT_F).astype(_f32)

    out = _k4(outp, bias_gat.reshape(1, OUT_F),
              W_ih.T, W_hh.T, b_ih.reshape(1, 3 * HID), b_hh.reshape(1, 3 * HID),
              W1, b1.reshape(1, 1), W2.T, b2.reshape(1, PRED))
    return (out.reshape(1, N, PRED), A_wave)


# speedup vs baseline: 46.4603x; 1.2304x over previous
import jax
import jax.numpy as jnp
from jax import lax
from jax.experimental import pallas as pl
from jax.experimental.pallas import tpu as pltpu
from jax.experimental.pallas import tpu_sc as plsc

N = 10000
E = 160000
EA = E + N
HIS = 12
HID = 16
HEADS = 3
OUT_F = 192
F = 576
HF = 640
PRED = 6

NC = 2
NS = 16
NW = NC * NS
CPT = 334
EPT = CPT * 16
EPAD = NW * EPT
ACCF = 30720
RPT = ACCF // NS

_f32 = jnp.float32
_bf16 = jnp.bfloat16
_i32 = jnp.int32

_SC_PARAMS = pltpu.CompilerParams(
    needs_layout_passes=False, use_tc_tiling_on_sc=False)


def _sc_mesh():
    return plsc.VectorSubcoreMesh(core_axis_name="c", subcore_axis_name="s")


def _k1_body(x_ref, w_ref, as_ref, ad_ref, h_ref, logit_ref):
    h = jnp.dot(x_ref[...], w_ref[...], preferred_element_type=_f32)
    h_ref[...] = h.astype(_bf16)
    h3 = h.reshape(h.shape[0], HEADS, OUT_F)
    a_s = jnp.sum(h3 * as_ref[...][None], axis=-1)
    a_d = jnp.sum(h3 * ad_ref[...][None], axis=-1)
    logit_ref[...] = jnp.concatenate([a_s, a_d], axis=-1)


def _k1(xf, w_gat, att_s, att_d):
    blk = 1000
    return pl.pallas_call(
        _k1_body,
        grid=(N // blk,),
        in_specs=[
            pl.BlockSpec((blk, 96), lambda i: (i, 0)),
            pl.BlockSpec((96, F), lambda i: (0, 0)),
            pl.BlockSpec((HEADS, OUT_F), lambda i: (0, 0)),
            pl.BlockSpec((HEADS, OUT_F), lambda i: (0, 0)),
        ],
        out_specs=[
            pl.BlockSpec((blk, F), lambda i: (i, 0)),
            pl.BlockSpec((blk, 6), lambda i: (i, 0)),
        ],
        out_shape=[
            jax.ShapeDtypeStruct((N, F), _bf16),
            jax.ShapeDtypeStruct((N, 6), _f32),
        ],
    )(xf, w_gat, att_s, att_d)


def _k2_body(src_hbm, dst_hbm, logit_hbm, wt_hbm, parts_hbm,
             srcv, dstv, adt, acc, wbuf):
    cid = lax.axis_index("c")
    sid = lax.axis_index("s")
    wid = sid * NC + cid
    base = wid * EPT

    pltpu.sync_copy(src_hbm.at[pl.ds(base, EPT)], srcv)
    pltpu.sync_copy(dst_hbm.at[pl.ds(base, EPT)], dstv)
    pltpu.sync_copy(logit_hbm, adt)

    def _zero(i, _):
        acc[pl.ds(i * 16, 16)] = jnp.zeros((16,), _f32)
        return _
    lax.fori_loop(0, ACCF // 16, _zero, None)

    iota = lax.iota(_i32, 16)

    def _chunk(c, _):
        off = c * 16
        s16 = srcv[pl.ds(off, 16)]
        d16 = dstv[pl.ds(off, 16)]
        valid = (base + off + iota) < EA
        s6 = s16 * 6
        d6 = d16 * 6
        for hh in range(HEADS):
            a = plsc.load_gather(adt, [s6 + hh]) + \
                plsc.load_gather(adt, [d6 + (hh + 3)])
            a = jnp.where(a > 0, a, a * 0.2)
            w = jnp.where(valid, jnp.exp(a), 0.0)
            wbuf[pl.ds(hh * EPT + off, 16)] = w
            plsc.addupdate_scatter(acc, [d16 * 3 + hh], w)
        return _
    lax.fori_loop(0, CPT, _chunk, None)

    for hh in range(HEADS):
        pltpu.sync_copy(wbuf.at[pl.ds(hh * EPT, EPT)],
                        wt_hbm.at[pl.ds(hh * EPAD + base, EPT)])
    pltpu.sync_copy(acc, parts_hbm.at[pl.ds(wid * ACCF, ACCF)])


def _k2(srcp, dstp, logit):
    fn = pl.kernel(
        _k2_body,
        out_type=[
            jax.ShapeDtypeStruct((HEADS * EPAD,), _f32),
            jax.ShapeDtypeStruct((NW * ACCF,), _f32),
        ],
        mesh=_sc_mesh(),
        scratch_types=[
            pltpu.VMEM((EPT,), _i32),
            pltpu.VMEM((EPT,), _i32),
            pltpu.VMEM((N * 6,), _f32),
            pltpu.VMEM((ACCF,), _f32),
            pltpu.VMEM((HEADS * EPT,), _f32),
        ],
        compiler_params=_SC_PARAMS,
    )
    return fn(srcp, dstp, logit)


def _k2b_body(dst_hbm, wt_hbm, parts_hbm, ct_hbm,
              dstv, wv, rdt, dsum, tmp, rdenS):
    cid = lax.axis_index("c")
    sid = lax.axis_index("s")
    wid = sid * NC + cid
    base = wid * EPT

    pltpu.sync_copy(dst_hbm.at[pl.ds(base, EPT)], dstv)
    for hh in range(HEADS):
        pltpu.sync_copy(wt_hbm.at[pl.ds(hh * EPAD + base, EPT)],
                        wv.at[pl.ds(hh * EPT, EPT)])

    fr = sid * RPT
    pltpu.sync_copy(parts_hbm.at[pl.ds(fr, RPT)], dsum)

    def _accum_part(w, _):
        pltpu.sync_copy(parts_hbm.at[pl.ds(w * ACCF + fr, RPT)], tmp)

        def _add(t, __):
            so = pl.ds(t * 16, 16)
            dsum[so] = dsum[so] + tmp[so]
            return __
        lax.fori_loop(0, RPT // 16, _add, None)
        return _
    lax.fori_loop(1, NW, _accum_part, None)

    def _recip(t, _):
        so = pl.ds(t * 16, 16)
        dsum[so] = 1.0 / (dsum[so] + 1e-16)
        return _
    lax.fori_loop(0, RPT // 16, _recip, None)
    pltpu.sync_copy(dsum, rdenS.at[pl.ds(fr, RPT)])
    plsc.subcore_barrier()
    pltpu.sync_copy(rdenS, rdt)

    third = 1.0 / HEADS

    def _chunk(c, _):
        off = c * 16
        d16 = dstv[pl.ds(off, 16)]
        d3 = d16 * 3
        for hh in range(HEADS):
            sl = pl.ds(hh * EPT + off, 16)
            r16 = plsc.load_gather(rdt, [d3 + hh])
            wv[sl] = wv[sl] * r16 * third
        return _
    lax.fori_loop(0, CPT, _chunk, None)

    for hh in range(HEADS):
        pltpu.sync_copy(wv.at[pl.ds(hh * EPT, EPT)],
                        ct_hbm.at[pl.ds(hh * EPAD + base, EPT)])


def _k2b(dstp, wt, parts):
    fn = pl.kernel(
        _k2b_body,
        out_type=jax.ShapeDtypeStruct((HEADS * EPAD,), _f32),
        mesh=_sc_mesh(),
        scratch_types=[
            pltpu.VMEM((EPT,), _i32),
            pltpu.VMEM((HEADS * EPT,), _f32),
            pltpu.VMEM((ACCF,), _f32),
            pltpu.VMEM((RPT,), _f32),
            pltpu.VMEM((RPT,), _f32),
            pltpu.MemorySpace.VMEM_SHARED((ACCF,), _f32),
        ],
        compiler_params=_SC_PARAMS,
    )
    return fn(dstp, wt, parts)


def _k3_body(src_hbm, dst_hbm, ct_hbm, h_hbm, outp_hbm,
             srcv, dstv, ctv, rows, msg, accum,
             gsem0, gsem1, ssem0, ssem1):
    cid = lax.axis_index("c")
    sid = lax.axis_index("s")
    wid = sid * NC + cid
    base = wid * EPT

    pltpu.sync_copy(src_hbm.at[pl.ds(base, EPT)], srcv)
    pltpu.sync_copy(dst_hbm.at[pl.ds(base, EPT)], dstv)
    for hh in range(HEADS):
        pltpu.sync_copy(ct_hbm.at[pl.ds(hh * EPAD + base, EPT)],
                        ctv.at[pl.ds(hh * EPT, EPT)])

    for j in range(16):
        for q in range(OUT_F // 32):
            z = jnp.zeros((32,), _bf16)
            msg[0, j, pl.ds(q * 32, 32)] = z
    nrows = 624
    for r in range(nrows // 16):
        pltpu.sync_copy(msg.at[0], accum.at[pl.ds(sid * nrows + r * 16, 16)])

    @pl.when(sid == NS - 1)
    def _zero_tail():
        pltpu.sync_copy(msg.at[0], accum.at[pl.ds(NS * nrows, 16)])

    plsc.subcore_barrier()

    def _chunk(c, b):
        gsem = gsem0 if b == 0 else gsem1
        ssem = ssem0 if b == 0 else ssem1
        s16 = srcv[pl.ds(c * 16, 16)]
        pltpu.make_async_copy(h_hbm.at[s16], rows.at[b], gsem).wait()
        d16 = dstv[pl.ds(c * 16, 16)]

        @pl.when(c >= 2)
        def _drain_prev_scatter():
            pltpu.make_async_copy(
                msg.at[b], accum.at[pl.ds(0, 16)], ssem).wait()

        cv = [ctv[pl.ds(hh * EPT + c * 16, 16)] for hh in range(HEADS)]
        for j in range(16):
            c0 = cv[0][j]
            c1 = cv[1][j]
            c2 = cv[2][j]
            for q in range(OUT_F // 32):
                r0e, r0o = plsc.unpack(
                    rows[b, j, pl.ds(q * 32, 32)],
                    format=plsc.PackFormat.INTERLEAVED)
                r1e, r1o = plsc.unpack(
                    rows[b, j, pl.ds(OUT_F + q * 32, 32)],
                    format=plsc.PackFormat.INTERLEAVED)
                r2e, r2o = plsc.unpack(
                    rows[b, j, pl.ds(2 * OUT_F + q * 32, 32)],
                    format=plsc.PackFormat.INTERLEAVED)
                me = r0e * c0 + r1e * c1 + r2e * c2
                mo = r0o * c0 + r1o * c1 + r2o * c2
                msg[b, j, pl.ds(q * 32, 32)] = plsc.pack(
                    me, mo, format=plsc.PackFormat.INTERLEAVED)

        @pl.when(c + 2 < CPT)
        def _issue_next():
            s16n = srcv[pl.ds((c + 2) * 16, 16)]
            pltpu.async_copy(h_hbm.at[s16n], rows.at[b], gsem)

        pltpu.async_copy(msg.at[b], accum.at[d16], ssem, add=True)

    s0 = srcv[pl.ds(0, 16)]
    pltpu.async_copy(h_hbm.at[s0], rows.at[0], gsem0)
    s1 = srcv[pl.ds(16, 16)]
    pltpu.async_copy(h_hbm.at[s1], rows.at[1], gsem1)

    def _pair(i, _):
        _chunk(2 * i, 0)
        _chunk(2 * i + 1, 1)
        return _
    lax.fori_loop(0, CPT // 2, _pair, None)

    pltpu.make_async_copy(msg.at[0], accum.at[pl.ds(0, 16)], ssem0).wait()
    pltpu.make_async_copy(msg.at[1], accum.at[pl.ds(0, 16)], ssem1).wait()

    plsc.subcore_barrier()
    pltpu.sync_copy(accum.at[pl.ds(sid * nrows, nrows)],
                    outp_hbm.at[pl.ds(cid * N + sid * nrows, nrows)])

    @pl.when(sid == NS - 1)
    def _out_tail():
        pltpu.sync_copy(accum.at[pl.ds(NS * nrows, 16)],
                        outp_hbm.at[pl.ds(cid * N + NS * nrows, 16)])


def _k3(srcp, dstp, ct, h):
    fn = pl.kernel(
        _k3_body,
        out_type=jax.ShapeDtypeStruct((NC * N, OUT_F), _bf16),
        mesh=_sc_mesh(),
        scratch_types=[
            pltpu.VMEM((EPT,), _i32),
            pltpu.VMEM((EPT,), _i32),
            pltpu.VMEM((HEADS * EPT,), _f32),
            pltpu.VMEM((2, 16, F), _bf16),
            pltpu.VMEM((2, 16, OUT_F), _bf16),
            pltpu.MemorySpace.VMEM_SHARED((N, OUT_F), _bf16),
            pltpu.SemaphoreType.DMA,
            pltpu.SemaphoreType.DMA,
            pltpu.SemaphoreType.DMA,
            pltpu.SemaphoreType.DMA,
        ],
        compiler_params=_SC_PARAMS,
    )
    return fn(srcp, dstp, ct, h)


def _k4_body(outp_ref, bias_ref, wih_ref, whh_ref, bih_ref, bhh_ref,
             w1_ref, b1_ref, w2_ref, b2_ref, out_ref):
    g = (outp_ref[0].astype(_f32) + outp_ref[1].astype(_f32)
         + bias_ref[...])
    hT = jnp.zeros((HID, N), _f32)
    accT = jnp.zeros((PRED, N), _f32)
    for t in range(HIS):
        xtT = g[t * HID:(t + 1) * HID, :]
        giT = jnp.dot(wih_ref[...], xtT, preferred_element_type=_f32) + bih_ref[...]
        ghT = jnp.dot(whh_ref[...], hT, preferred_element_type=_f32) + bhh_ref[...]
        r = jax.nn.sigmoid(giT[:HID] + ghT[:HID])
        z = jax.nn.sigmoid(giT[HID:2 * HID] + ghT[HID:2 * HID])
        nn_ = jnp.tanh(giT[2 * HID:] + r * ghT[2 * HID:])
        hT = (1.0 - z) * nn_ + z * hT
        o1 = jnp.dot(w1_ref[...], hT, preferred_element_type=_f32) + b1_ref[...]
        accT = accT + o1 * w2_ref[:, t:t + 1]
    out_ref[...] = accT + b2_ref[...]


def _k4(outpT, biasT, wih, whh, bihT, bhhT, w1, b1, w2, b2T):
    return pl.pallas_call(
        _k4_body,
        grid=(1,),
        in_specs=[
            pl.BlockSpec((NC, OUT_F, N), lambda i: (0, 0, 0)),
            pl.BlockSpec((OUT_F, 1), lambda i: (0, 0)),
            pl.BlockSpec((3 * HID, HID), lambda i: (0, 0)),
            pl.BlockSpec((3 * HID, HID), lambda i: (0, 0)),
            pl.BlockSpec((3 * HID, 1), lambda i: (0, 0)),
            pl.BlockSpec((3 * HID, 1), lambda i: (0, 0)),
            pl.BlockSpec((1, HID), lambda i: (0, 0)),
            pl.BlockSpec((1, 1), lambda i: (0, 0)),
            pl.BlockSpec((PRED, HIS), lambda i: (0, 0)),
            pl.BlockSpec((PRED, 1), lambda i: (0, 0)),
        ],
        out_specs=pl.BlockSpec((PRED, N), lambda i: (0, 0)),
        out_shape=jax.ShapeDtypeStruct((PRED, N), _f32),
    )(outpT, biasT, wih, whh, bihT, bhhT, w1, b1, w2, b2T)


def kernel(x, edge_index, A_wave, W_gat, att_src, att_dst, bias_gat,
           W_ih, W_hh, b_ih, b_hh, W1, b1, W2, b2):
    xf = x.reshape(N, 96)
    loop = jnp.arange(N, dtype=edge_index.dtype)
    padz = jnp.zeros((EPAD - EA,), edge_index.dtype)
    srcp = jnp.concatenate([edge_index[0], loop, padz])
    dstp = jnp.concatenate([edge_index[1], loop, padz])

    h, logit = _k1(xf, W_gat,
                   att_src.reshape(HEADS, OUT_F),
                   att_dst.reshape(HEADS, OUT_F))
    wt, parts = _k2(srcp, dstp, logit.reshape(N * 6))
    ct = _k2b(dstp, wt, parts)
    outp = _k3(srcp, dstp, ct, h)
    outpT = jnp.swapaxes(outp.reshape(NC, N, OUT_F), 1, 2)

    outT = _k4(outpT, bias_gat.reshape(OUT_F, 1),
               W_ih, W_hh, b_ih.reshape(3 * HID, 1), b_hh.reshape(3 * HID, 1),
               W1, b1.reshape(1, 1), W2, b2.reshape(PRED, 1))
    return (outT.T.reshape(1, N, PRED), A_wave)


# speedup vs baseline: 46.5578x; 1.0021x over previous
import jax
import jax.numpy as jnp
from jax import lax
from jax.experimental import pallas as pl
from jax.experimental.pallas import tpu as pltpu
from jax.experimental.pallas import tpu_sc as plsc

N = 10000
E = 160000
EA = E + N
HIS = 12
HID = 16
HEADS = 3
OUT_F = 192
F = 576
HF = 640
PRED = 6

NC = 2
NS = 16
NW = NC * NS
CPT = 334
EPT = CPT * 16
EPAD = NW * EPT
ACCF = 30720
RPT = ACCF // NS

_f32 = jnp.float32
_bf16 = jnp.bfloat16
_i32 = jnp.int32

_SC_PARAMS = pltpu.CompilerParams(
    needs_layout_passes=False, use_tc_tiling_on_sc=False)


def _sc_mesh():
    return plsc.VectorSubcoreMesh(core_axis_name="c", subcore_axis_name="s")


def _k1_body(x_ref, w_ref, as_ref, ad_ref, h_ref, logit_ref):
    h = jnp.dot(x_ref[...], w_ref[...], preferred_element_type=_f32)
    h_ref[...] = h.astype(_bf16)
    h3 = h.reshape(h.shape[0], HEADS, OUT_F)
    a_s = jnp.sum(h3 * as_ref[...][None], axis=-1)
    a_d = jnp.sum(h3 * ad_ref[...][None], axis=-1)
    logit_ref[...] = jnp.concatenate([a_s, a_d], axis=-1)


def _k1(xf, w_gat, att_s, att_d):
    blk = 1000
    return pl.pallas_call(
        _k1_body,
        grid=(N // blk,),
        in_specs=[
            pl.BlockSpec((blk, 96), lambda i: (i, 0)),
            pl.BlockSpec((96, F), lambda i: (0, 0)),
            pl.BlockSpec((HEADS, OUT_F), lambda i: (0, 0)),
            pl.BlockSpec((HEADS, OUT_F), lambda i: (0, 0)),
        ],
        out_specs=[
            pl.BlockSpec((blk, F), lambda i: (i, 0)),
            pl.BlockSpec((blk, 6), lambda i: (i, 0)),
        ],
        out_shape=[
            jax.ShapeDtypeStruct((N, F), _bf16),
            jax.ShapeDtypeStruct((N, 6), _f32),
        ],
    )(xf, w_gat, att_s, att_d)


def _k2_body(src_hbm, dst_hbm, logit_hbm, wt_hbm, parts_hbm,
             srcv, dstv, adt, acc, wbuf):
    cid = lax.axis_index("c")
    sid = lax.axis_index("s")
    wid = sid * NC + cid
    base = wid * EPT

    pltpu.sync_copy(src_hbm.at[pl.ds(base, EPT)], srcv)
    pltpu.sync_copy(dst_hbm.at[pl.ds(base, EPT)], dstv)
    pltpu.sync_copy(logit_hbm, adt)

    def _zero(i, _):
        acc[pl.ds(i * 16, 16)] = jnp.zeros((16,), _f32)
        return _
    lax.fori_loop(0, ACCF // 16, _zero, None)

    iota = lax.iota(_i32, 16)

    def _chunk(c, _):
        off = c * 16
        s16 = srcv[pl.ds(off, 16)]
        d16 = dstv[pl.ds(off, 16)]
        valid = (base + off + iota) < EA
        s6 = s16 * 6
        d6 = d16 * 6
        for hh in range(HEADS):
            a = plsc.load_gather(adt, [s6 + hh]) + \
                plsc.load_gather(adt, [d6 + (hh + 3)])
            a = jnp.where(a > 0, a, a * 0.2)
            w = jnp.where(valid, jnp.exp(a), 0.0)
            wbuf[pl.ds(hh * EPT + off, 16)] = w
            plsc.addupdate_scatter(acc, [d16 * 3 + hh], w)
        return _
    lax.fori_loop(0, CPT, _chunk, None)

    for hh in range(HEADS):
        pltpu.sync_copy(wbuf.at[pl.ds(hh * EPT, EPT)],
                        wt_hbm.at[pl.ds(hh * EPAD + base, EPT)])
    pltpu.sync_copy(acc, parts_hbm.at[pl.ds(wid * ACCF, ACCF)])


def _k2(srcp, dstp, logit):
    fn = pl.kernel(
        _k2_body,
        out_type=[
            jax.ShapeDtypeStruct((HEADS * EPAD,), _f32),
            jax.ShapeDtypeStruct((NW * ACCF,), _f32),
        ],
        mesh=_sc_mesh(),
        scratch_types=[
            pltpu.VMEM((EPT,), _i32),
            pltpu.VMEM((EPT,), _i32),
            pltpu.VMEM((N * 6,), _f32),
            pltpu.VMEM((ACCF,), _f32),
            pltpu.VMEM((HEADS * EPT,), _f32),
        ],
        compiler_params=_SC_PARAMS,
    )
    return fn(srcp, dstp, logit)


def _k2b_body(dst_hbm, wt_hbm, parts_hbm, ct_hbm,
              dstv, wv, rdt, dsum, tmp, cbuf, rdenS):
    cid = lax.axis_index("c")
    sid = lax.axis_index("s")
    wid = sid * NC + cid
    base = wid * EPT

    pltpu.sync_copy(dst_hbm.at[pl.ds(base, EPT)], dstv)
    for hh in range(HEADS):
        pltpu.sync_copy(wt_hbm.at[pl.ds(hh * EPAD + base, EPT)],
                        wv.at[pl.ds(hh * EPT, EPT)])

    fr = sid * RPT
    pltpu.sync_copy(parts_hbm.at[pl.ds(fr, RPT)], dsum)

    def _accum_part(w, _):
        pltpu.sync_copy(parts_hbm.at[pl.ds(w * ACCF + fr, RPT)], tmp)

        def _add(t, __):
            so = pl.ds(t * 16, 16)
            dsum[so] = dsum[so] + tmp[so]
            return __
        lax.fori_loop(0, RPT // 16, _add, None)
        return _
    lax.fori_loop(1, NW, _accum_part, None)

    def _recip(t, _):
        so = pl.ds(t * 16, 16)
        dsum[so] = 1.0 / (dsum[so] + 1e-16)
        return _
    lax.fori_loop(0, RPT // 16, _recip, None)
    pltpu.sync_copy(dsum, rdenS.at[pl.ds(fr, RPT)])
    plsc.subcore_barrier()
    pltpu.sync_copy(rdenS, rdt)

    third = 1.0 / HEADS

    def _chunk(c, _):
        off = c * 16
        d16 = dstv[pl.ds(off, 16)]
        d3 = d16 * 3
        for hh in range(HEADS):
            sl = pl.ds(hh * EPT + off, 16)
            r16 = plsc.load_gather(rdt, [d3 + hh])
            cf = wv[sl] * r16 * third
            cbuf[pl.ds(hh * EPT * 2 + off * 2, 32)] = plsc.pack(
                cf, cf, format=plsc.PackFormat.INTERLEAVED)
        return _
    lax.fori_loop(0, CPT, _chunk, None)

    for hh in range(HEADS):
        pltpu.sync_copy(cbuf.at[pl.ds(hh * EPT * 2, EPT * 2)],
                        ct_hbm.at[pl.ds(hh * EPAD * 2 + base * 2, EPT * 2)])


def _k2b(dstp, wt, parts):
    fn = pl.kernel(
        _k2b_body,
        out_type=jax.ShapeDtypeStruct((HEADS * EPAD * 2,), _bf16),
        mesh=_sc_mesh(),
        scratch_types=[
            pltpu.VMEM((EPT,), _i32),
            pltpu.VMEM((HEADS * EPT,), _f32),
            pltpu.VMEM((ACCF,), _f32),
            pltpu.VMEM((RPT,), _f32),
            pltpu.VMEM((RPT,), _f32),
            pltpu.VMEM((HEADS * EPT * 2,), _bf16),
            pltpu.MemorySpace.VMEM_SHARED((ACCF,), _f32),
        ],
        compiler_params=_SC_PARAMS,
    )
    return fn(dstp, wt, parts)


def _k3_body(src_hbm, dst_hbm, ct_hbm, h_hbm, outp_hbm,
             srcv, dstv, ctv, rows, msg, accum,
             gsem0, gsem1, ssem0, ssem1):
    cid = lax.axis_index("c")
    sid = lax.axis_index("s")
    wid = sid * NC + cid
    base = wid * EPT

    pltpu.sync_copy(src_hbm.at[pl.ds(base, EPT)], srcv)
    pltpu.sync_copy(dst_hbm.at[pl.ds(base, EPT)], dstv)
    for hh in range(HEADS):
        pltpu.sync_copy(ct_hbm.at[pl.ds(hh * EPAD * 2 + base * 2, EPT * 2)],
                        ctv.at[pl.ds(hh * EPT * 2, EPT * 2)])

    for j in range(16):
        for q in range(OUT_F // 32):
            z = jnp.zeros((32,), _bf16)
            msg[0, j, pl.ds(q * 32, 32)] = z
    nrows = 624
    for r in range(nrows // 16):
        pltpu.sync_copy(msg.at[0], accum.at[pl.ds(sid * nrows + r * 16, 16)])

    @pl.when(sid == NS - 1)
    def _zero_tail():
        pltpu.sync_copy(msg.at[0], accum.at[pl.ds(NS * nrows, 16)])

    plsc.subcore_barrier()

    def _chunk(c, b):
        gsem = gsem0 if b == 0 else gsem1
        ssem = ssem0 if b == 0 else ssem1
        s16 = srcv[pl.ds(c * 16, 16)]
        pltpu.make_async_copy(h_hbm.at[s16], rows.at[b], gsem).wait()
        d16 = dstv[pl.ds(c * 16, 16)]

        @pl.when(c >= 2)
        def _drain_prev_scatter():
            pltpu.make_async_copy(
                msg.at[b], accum.at[pl.ds(0, 16)], ssem).wait()

        cv = [plsc.bitcast(ctv[pl.ds(hh * EPT * 2 + c * 32, 32)], _i32)
              for hh in range(HEADS)]
        for j in range(16):
            c0 = plsc.bitcast(jnp.broadcast_to(cv[0][j], (16,)), _bf16)
            c1 = plsc.bitcast(jnp.broadcast_to(cv[1][j], (16,)), _bf16)
            c2 = plsc.bitcast(jnp.broadcast_to(cv[2][j], (16,)), _bf16)
            for q in range(OUT_F // 32):
                msg[b, j, pl.ds(q * 32, 32)] = (
                    rows[b, j, pl.ds(q * 32, 32)] * c0
                    + rows[b, j, pl.ds(OUT_F + q * 32, 32)] * c1
                    + rows[b, j, pl.ds(2 * OUT_F + q * 32, 32)] * c2)

        @pl.when(c + 2 < CPT)
        def _issue_next():
            s16n = srcv[pl.ds((c + 2) * 16, 16)]
            pltpu.async_copy(h_hbm.at[s16n], rows.at[b], gsem)

        pltpu.async_copy(msg.at[b], accum.at[d16], ssem, add=True)

    s0 = srcv[pl.ds(0, 16)]
    pltpu.async_copy(h_hbm.at[s0], rows.at[0], gsem0)
    s1 = srcv[pl.ds(16, 16)]
    pltpu.async_copy(h_hbm.at[s1], rows.at[1], gsem1)

    def _pair(i, _):
        _chunk(2 * i, 0)
        _chunk(2 * i + 1, 1)
        return _
    lax.fori_loop(0, CPT // 2, _pair, None)

    pltpu.make_async_copy(msg.at[0], accum.at[pl.ds(0, 16)], ssem0).wait()
    pltpu.make_async_copy(msg.at[1], accum.at[pl.ds(0, 16)], ssem1).wait()

    plsc.subcore_barrier()
    pltpu.sync_copy(accum.at[pl.ds(sid * nrows, nrows)],
                    outp_hbm.at[pl.ds(cid * N + sid * nrows, nrows)])

    @pl.when(sid == NS - 1)
    def _out_tail():
        pltpu.sync_copy(accum.at[pl.ds(NS * nrows, 16)],
                        outp_hbm.at[pl.ds(cid * N + NS * nrows, 16)])


def _k3(srcp, dstp, ct, h):
    fn = pl.kernel(
        _k3_body,
        out_type=jax.ShapeDtypeStruct((NC * N, OUT_F), _bf16),
        mesh=_sc_mesh(),
        scratch_types=[
            pltpu.VMEM((EPT,), _i32),
            pltpu.VMEM((EPT,), _i32),
            pltpu.VMEM((HEADS * EPT * 2,), _bf16),
            pltpu.VMEM((2, 16, F), _bf16),
            pltpu.VMEM((2, 16, OUT_F), _bf16),
            pltpu.MemorySpace.VMEM_SHARED((N, OUT_F), _bf16),
            pltpu.SemaphoreType.DMA,
            pltpu.SemaphoreType.DMA,
            pltpu.SemaphoreType.DMA,
            pltpu.SemaphoreType.DMA,
        ],
        compiler_params=_SC_PARAMS,
    )
    return fn(srcp, dstp, ct, h)


def _k4_body(outp_ref, bias_ref, wih_ref, whh_ref, bih_ref, bhh_ref,
             w1_ref, b1_ref, w2_ref, b2_ref, out_ref):
    g = (outp_ref[0].astype(_f32) + outp_ref[1].astype(_f32)
         + bias_ref[...])
    hT = jnp.zeros((HID, N), _f32)
    accT = jnp.zeros((PRED, N), _f32)
    for t in range(HIS):
        xtT = g[t * HID:(t + 1) * HID, :]
        giT = jnp.dot(wih_ref[...], xtT, preferred_element_type=_f32) + bih_ref[...]
        ghT = jnp.dot(whh_ref[...], hT, preferred_element_type=_f32) + bhh_ref[...]
        r = jax.nn.sigmoid(giT[:HID] + ghT[:HID])
        z = jax.nn.sigmoid(giT[HID:2 * HID] + ghT[HID:2 * HID])
        nn_ = jnp.tanh(giT[2 * HID:] + r * ghT[2 * HID:])
        hT = (1.0 - z) * nn_ + z * hT
        o1 = jnp.dot(w1_ref[...], hT, preferred_element_type=_f32) + b1_ref[...]
        accT = accT + o1 * w2_ref[:, t:t + 1]
    out_ref[...] = accT + b2_ref[...]


def _k4(outpT, biasT, wih, whh, bihT, bhhT, w1, b1, w2, b2T):
    return pl.pallas_call(
        _k4_body,
        grid=(1,),
        in_specs=[
            pl.BlockSpec((NC, OUT_F, N), lambda i: (0, 0, 0)),
            pl.BlockSpec((OUT_F, 1), lambda i: (0, 0)),
            pl.BlockSpec((3 * HID, HID), lambda i: (0, 0)),
            pl.BlockSpec((3 * HID, HID), lambda i: (0, 0)),
            pl.BlockSpec((3 * HID, 1), lambda i: (0, 0)),
            pl.BlockSpec((3 * HID, 1), lambda i: (0, 0)),
            pl.BlockSpec((1, HID), lambda i: (0, 0)),
            pl.BlockSpec((1, 1), lambda i: (0, 0)),
            pl.BlockSpec((PRED, HIS), lambda i: (0, 0)),
            pl.BlockSpec((PRED, 1), lambda i: (0, 0)),
        ],
        out_specs=pl.BlockSpec((PRED, N), lambda i: (0, 0)),
        out_shape=jax.ShapeDtypeStruct((PRED, N), _f32),
    )(outpT, biasT, wih, whh, bihT, bhhT, w1, b1, w2, b2T)


def kernel(x, edge_index, A_wave, W_gat, att_src, att_dst, bias_gat,
           W_ih, W_hh, b_ih, b_hh, W1, b1, W2, b2):
    xf = x.reshape(N, 96)
    loop = jnp.arange(N, dtype=edge_index.dtype)
    padz = jnp.zeros((EPAD - EA,), edge_index.dtype)
    srcp = jnp.concatenate([edge_index[0], loop, padz])
    dstp = jnp.concatenate([edge_index[1], loop, padz])

    h, logit = _k1(xf, W_gat,
                   att_src.reshape(HEADS, OUT_F),
                   att_dst.reshape(HEADS, OUT_F))
    wt, parts = _k2(srcp, dstp, logit.reshape(N * 6))
    ct = _k2b(dstp, wt, parts)
    outp = _k3(srcp, dstp, ct, h)
    outpT = jnp.swapaxes(outp.reshape(NC, N, OUT_F), 1, 2)

    outT = _k4(outpT, bias_gat.reshape(OUT_F, 1),
               W_ih, W_hh, b_ih.reshape(3 * HID, 1), b_hh.reshape(3 * HID, 1),
               W1, b1.reshape(1, 1), W2, b2.reshape(PRED, 1))
    return (outT.T.reshape(1, N, PRED), A_wave)
